# Initial kernel scaffold; baseline (speedup 1.0000x reference)
import jax
import jax.numpy as jnp
from jax.experimental import pallas as pl


def kernel(x, edge_index, weight, bias):
    raise NotImplementedError("write your pallas kernel here")


# speedup vs baseline: 6.6346x; 6.6346x over previous
import functools

import jax
import jax.numpy as jnp
from jax import lax
from jax.experimental import pallas as pl
from jax.experimental.pallas import tpu as pltpu
from jax.experimental.pallas import tpu_sc as plsc

_N = 10000
_NP = 10240
_E = 320000
_EP = 327680
_C = 128
_TRASH = _N
_NW = 32
_EPW = _EP // _NW
_SL = _NP // 16
_F32 = jnp.float32

_mesh = plsc.VectorSubcoreMesh(core_axis_name="c", subcore_axis_name="s")


def _dot(a, b):
    return lax.dot_general(
        a, b, (((1,), (0,)), ((), ())),
        precision=lax.Precision.HIGHEST, preferred_element_type=_F32)


@functools.partial(
    pl.kernel,
    out_type=jax.ShapeDtypeStruct((2, _NP, _C), _F32),
    mesh=_mesh,
    scratch_types=[
        pltpu.VMEM_SHARED((_NP, _C), _F32),
        pltpu.VMEM((128, _C), _F32),
        pltpu.VMEM((4, 128), jnp.int32),
        pltpu.VMEM((4, 128), jnp.int32),
        pltpu.VMEM((4, 128), jnp.int32),
    ],
)
def _deg_kernel(rows_hbm, cols_hbm, zrow_hbm, deg_hbm,
                acc, obuf, rbuf, cbuf, mbuf):
    c = lax.axis_index("c")
    s = lax.axis_index("s")
    wid = c * 16 + s
    ones16 = jnp.ones((16,), _F32)
    for i in range(128):
        for l in range(8):
            obuf[i, pl.ds(l * 16, 16)] = ones16
    pltpu.sync_copy(zrow_hbm, acc.at[pl.ds(s * _SL, _SL)])
    plsc.subcore_barrier()

    @pl.loop(0, _EPW // 512)
    def _(t):
        base = wid * 80 + t * 4
        pltpu.sync_copy(rows_hbm.at[pl.ds(base, 4)], rbuf)
        pltpu.sync_copy(cols_hbm.at[pl.ds(base, 4)], cbuf)
        for j in range(4):
            for l in range(8):
                rv = rbuf[j, pl.ds(l * 16, 16)]
                cv = cbuf[j, pl.ds(l * 16, 16)]
                mbuf[j, pl.ds(l * 16, 16)] = jnp.where(rv == cv, _TRASH, rv)
        for j in range(4):
            pltpu.sync_copy(obuf, acc.at[mbuf.at[j]], add=True)

    plsc.subcore_barrier()
    pltpu.sync_copy(acc.at[pl.ds(s * _SL, _SL)],
                    deg_hbm.at[c].at[pl.ds(s * _SL, _SL)])


@functools.partial(
    pl.kernel,
    out_type=jax.ShapeDtypeStruct((2, _NP, _C), _F32),
    mesh=_mesh,
    scratch_types=[
        pltpu.VMEM_SHARED((_NP, _C), _F32),
        pltpu.VMEM((4, 128), jnp.int32),
        pltpu.VMEM((4, 128), jnp.int32),
        pltpu.VMEM((4, 128), jnp.int32),
        pltpu.VMEM((2, 128, _C), _F32),
        pltpu.SemaphoreType.DMA,
        pltpu.SemaphoreType.DMA,
    ],
)
def _spmm_kernel(table_hbm, cols_hbm, rows_hbm, zrow_hbm, part_hbm,
                 acc, cbuf, rbuf, mbuf, gbuf, sem0, sem1):
    c = lax.axis_index("c")
    s = lax.axis_index("s")
    wid = c * 16 + s
    sems = (sem0, sem1)
    pltpu.sync_copy(zrow_hbm, acc.at[pl.ds(s * _SL, _SL)])
    plsc.subcore_barrier()

    @pl.loop(0, _EPW // 512)
    def _(t):
        base = wid * 80 + t * 4
        pltpu.sync_copy(cols_hbm.at[pl.ds(base, 4)], cbuf)
        pltpu.sync_copy(rows_hbm.at[pl.ds(base, 4)], rbuf)
        for j in range(4):
            for l in range(8):
                rv = rbuf[j, pl.ds(l * 16, 16)]
                cv = cbuf[j, pl.ds(l * 16, 16)]
                mbuf[j, pl.ds(l * 16, 16)] = jnp.where(rv == cv, _TRASH, rv)
        descs = [
            pltpu.async_copy(table_hbm.at[cbuf.at[0]], gbuf.at[0], sem0),
            pltpu.async_copy(table_hbm.at[cbuf.at[1]], gbuf.at[1], sem1),
        ]
        for j in range(4):
            b = j % 2
            descs[b].wait()
            pltpu.sync_copy(gbuf.at[b], acc.at[mbuf.at[j]], add=True)
            if j + 2 < 4:
                descs[b] = pltpu.async_copy(
                    table_hbm.at[cbuf.at[j + 2]], gbuf.at[b], sems[b])

    plsc.subcore_barrier()
    pltpu.sync_copy(acc.at[pl.ds(s * _SL, _SL)],
                    part_hbm.at[c].at[pl.ds(s * _SL, _SL)])


def _tc_out0(xp, w0, w2, b2d):
    def body(x_ref, w0_ref, w2_ref, b_ref, o_ref):
        o_ref[...] = _dot(x_ref[...], w0_ref[...] - w2_ref[...]) + b_ref[...]

    return pl.pallas_call(
        body,
        grid=(10,),
        in_specs=[
            pl.BlockSpec((1024, _C), lambda i: (i, 0)),
            pl.BlockSpec((_C, _C), lambda i: (0, 0)),
            pl.BlockSpec((_C, _C), lambda i: (0, 0)),
            pl.BlockSpec((1, _C), lambda i: (0, 0)),
        ],
        out_specs=pl.BlockSpec((1024, _C), lambda i: (i, 0)),
        out_shape=jax.ShapeDtypeStruct((_NP, _C), _F32),
    )(xp, w0, w2, b2d)


def _tc_scale1(degp, xp):
    def body(d_ref, x_ref, dinv_ref, xs_ref):
        deg = d_ref[0, :, 0:1] + d_ref[1, :, 0:1]
        dinv = jnp.where(deg > 0.0, lax.rsqrt(deg), 0.0)
        dinv_ref[...] = dinv
        xs_ref[...] = dinv * x_ref[...]

    return pl.pallas_call(
        body,
        grid=(10,),
        in_specs=[
            pl.BlockSpec((2, 1024, _C), lambda i: (0, i, 0)),
            pl.BlockSpec((1024, _C), lambda i: (i, 0)),
        ],
        out_specs=[
            pl.BlockSpec((1024, 1), lambda i: (i, 0)),
            pl.BlockSpec((1024, _C), lambda i: (i, 0)),
        ],
        out_shape=[
            jax.ShapeDtypeStruct((_NP, 1), _F32),
            jax.ShapeDtypeStruct((_NP, _C), _F32),
        ],
    )(degp, xp)


def _tc_comb1(part, dinv2, out0, w1):
    def body(p_ref, d_ref, o0_ref, w_ref, o1_ref, y_ref):
        u = p_ref[0] + p_ref[1]
        d = d_ref[...]
        du = d * u
        o1_ref[...] = o0_ref[...] - _dot(du, w_ref[...])
        y_ref[...] = d * du

    return pl.pallas_call(
        body,
        grid=(10,),
        in_specs=[
            pl.BlockSpec((2, 1024, _C), lambda i: (0, i, 0)),
            pl.BlockSpec((1024, 1), lambda i: (i, 0)),
            pl.BlockSpec((1024, _C), lambda i: (i, 0)),
            pl.BlockSpec((_C, _C), lambda i: (0, 0)),
        ],
        out_specs=[
            pl.BlockSpec((1024, _C), lambda i: (i, 0)),
            pl.BlockSpec((1024, _C), lambda i: (i, 0)),
        ],
        out_shape=[
            jax.ShapeDtypeStruct((_NP, _C), _F32),
            jax.ShapeDtypeStruct((_NP, _C), _F32),
        ],
    )(part, dinv2, out0, w1)


def _tc_comb2(part, dinv2, out1, w2):
    def body(p_ref, d_ref, o1_ref, w_ref, o_ref):
        u = p_ref[0] + p_ref[1]
        du = d_ref[...] * u
        o_ref[...] = o1_ref[...] + 2.0 * _dot(du, w_ref[...])

    return pl.pallas_call(
        body,
        grid=(10,),
        in_specs=[
            pl.BlockSpec((2, 1000, _C), lambda i: (0, i, 0)),
            pl.BlockSpec((1000, 1), lambda i: (i, 0)),
            pl.BlockSpec((1000, _C), lambda i: (i, 0)),
            pl.BlockSpec((_C, _C), lambda i: (0, 0)),
        ],
        out_specs=pl.BlockSpec((1000, _C), lambda i: (i, 0)),
        out_shape=jax.ShapeDtypeStruct((_N, _C), _F32),
    )(part, dinv2, out1, w2)


def kernel(x, edge_index, weight, bias):
    xp = jnp.pad(x[0], ((0, _NP - _N), (0, 0)))
    ei = edge_index.astype(jnp.int32)
    rows = jnp.pad(ei[0], (0, _EP - _E)).reshape(_EP // 128, 128)
    cols = jnp.pad(ei[1], (0, _EP - _E)).reshape(_EP // 128, 128)
    zrow = jnp.zeros((_SL, _C), _F32)
    b2d = bias.reshape(1, _C)

    degp = _deg_kernel(rows, cols, zrow)
    out0 = _tc_out0(xp, weight[0], weight[2], b2d)
    dinv2, xs = _tc_scale1(degp, xp)
    part1 = _spmm_kernel(xs, cols, rows, zrow)
    out1, ytab = _tc_comb1(part1, dinv2, out0, weight[1])
    part2 = _spmm_kernel(ytab, cols, rows, zrow)
    out = _tc_comb2(part2, dinv2, out1, weight[2])
    return out[None]
